# baseline (speedup 1.0000x reference)
import functools

import jax
import jax.numpy as jnp
from jax import lax
from jax.experimental import pallas as pl
from jax.experimental.pallas import tpu as pltpu
from jax.experimental.pallas import tpu_sc as plsc

N = 10000
E = 320000
D = 128
H1 = 4
HID = 128
DOC = 256
B = 16
OUT = 10

NC, NS, LANES = 2, 16, 16
NW = NC * NS
NPAD = 10240
EK = 32
SK = 8
SKE = SK * EK
ETOT = E + N
NCH = 328
NSUP = NCH // SK
EPAD = NW * NCH * EK
RPT = NPAD // NS
DR = NPAD // HID
TN = 256
NT = NPAD // TN


def _gat_sc_pass(H):
    mesh = plsc.VectorSubcoreMesh(
        core_axis_name="c", subcore_axis_name="s",
        num_cores=NC, num_subcores=NS)

    @functools.partial(
        pl.kernel,
        out_type=[jax.ShapeDtypeStruct((NC, H, NPAD, HID), jnp.float32),
                  jax.ShapeDtypeStruct((NC, H, DR, HID), jnp.float32)],
        mesh=mesh,
        scratch_types=[
            pltpu.VMEM((EK, HID), jnp.float32),
            pltpu.VMEM((EK, HID), jnp.float32),
            pltpu.VMEM((EK, HID), jnp.float32),
            pltpu.VMEM((EK, HID), jnp.float32),
            pltpu.VMEM((EK, HID), jnp.float32),
            pltpu.VMEM((EK, HID), jnp.float32),
            pltpu.VMEM((2, SKE), jnp.int32),
            pltpu.VMEM((2, SKE), jnp.int32),
            pltpu.VMEM((EK,), jnp.int32),
            pltpu.VMEM((EK,), jnp.int32),
            pltpu.VMEM((EK,), jnp.int32),
            pltpu.VMEM((EK,), jnp.int32),
            pltpu.VMEM((EK,), jnp.int32),
            pltpu.VMEM((EK,), jnp.int32),
            pltpu.VMEM((EK,), jnp.int32),
            pltpu.VMEM((EK,), jnp.int32),
            pltpu.VMEM((HID,), jnp.float32),
            pltpu.VMEM((DR, HID), jnp.float32),
            pltpu.VMEM((DR, HID), jnp.float32),
            pltpu.VMEM((DR,), jnp.int32),
            pltpu.VMEM_SHARED((NPAD, HID), jnp.float32),
            pltpu.VMEM_SHARED((DR, HID), jnp.float32),
            pltpu.SemaphoreType.DMA,
            pltpu.SemaphoreType.DMA,
            pltpu.SemaphoreType.DMA,
            pltpu.SemaphoreType.DMA,
            pltpu.SemaphoreType.DMA,
        ],
    )
    def gat_pass(xl_hbm, xr_hbm, att_hbm, srcs_hbm, dsts_hbm,
                 msg_out, den_out,
                 xl0, xl1, xr0, xr1, msg0, msg1, si, di,
                 svo0, svo1, dvo0, dvo1, dvb0, dvb1, dsc0, dsc1,
                 attv, den0, den1, riota, acc, dacc,
                 semg0, semg1, semsc0, semsc1, semi):
        cid = lax.axis_index("c")
        sid = lax.axis_index("s")
        wid = cid * NS + sid
        nslice = HID // LANES
        nsub = EK // LANES

        def ziota(r, car):
            riota[pl.ds(r * LANES, LANES)] = (
                lax.iota(jnp.int32, LANES) + r * LANES)
            return car
        lax.fori_loop(0, DR // LANES, ziota, 0)

        def head_body(h, hcar):
            def zrow(r, car):
                for c in range(nslice):
                    msg0[r, pl.ds(c * LANES, LANES)] = jnp.zeros(
                        (LANES,), jnp.float32)
                return car
            lax.fori_loop(0, EK, zrow, 0)

            def zden(r, car):
                for c in range(nslice):
                    z16 = jnp.zeros((LANES,), jnp.float32)
                    den0[r, pl.ds(c * LANES, LANES)] = z16
                    den1[r, pl.ds(c * LANES, LANES)] = z16
                return car
            lax.fori_loop(0, DR, zden, 0)

            base = pl.multiple_of(sid * RPT, 64)
            for k in range(RPT // EK):
                pltpu.sync_copy(msg0, acc.at[pl.ds(base + k * EK, EK)])

            @pl.when(sid < DR // 8)
            def _():
                d0 = pl.multiple_of(sid * 8, 8)
                pltpu.sync_copy(den0.at[pl.ds(0, 8)], dacc.at[pl.ds(d0, 8)])
            plsc.subcore_barrier()

            pltpu.sync_copy(att_hbm.at[h], attv)
            att_r = [attv[pl.ds(c * LANES, LANES)] for c in range(nslice)]

            pltpu.async_copy(srcs_hbm.at[wid, 0], si.at[0], semi)
            pltpu.async_copy(dsts_hbm.at[wid, 0], di.at[0], semi)

            def issue(ch, xlb, xrb, svob, dvob, dvbb, semg):
                sp = (ch >> 3) & 1
                cw = ch & (SK - 1)
                off = jnp.full((LANES,), h * NPAD, jnp.int32)
                for c in range(nsub):
                    sl = pl.ds(c * LANES, LANES)
                    s0 = pl.multiple_of(cw * EK + c * LANES, LANES)
                    sv = si[sp, pl.ds(s0, LANES)]
                    dv = di[sp, pl.ds(s0, LANES)]
                    svob[sl] = sv + off
                    dvob[sl] = dv + off
                    dvbb[sl] = dv
                c1 = pltpu.async_copy(xl_hbm.at[svob], xlb, semg)
                c2 = pltpu.async_copy(xr_hbm.at[dvob], xrb, semg)

            def compute(xlb, xrb, dvbb, msgb, dscb):
                def group(g, gcar):
                    dvv = dvbb[pl.ds(g * LANES, LANES)]
                    lanes = lax.iota(jnp.int32, LANES)
                    dscb[pl.ds(g * LANES, LANES)] = dvv
                    for eo in range(LANES):
                        e = g * LANES + eo
                        a = [xlb[e, pl.ds(c * LANES, LANES)]
                             for c in range(nslice)]
                        accv = None
                        for c in range(nslice):
                            z = a[c] + xrb[e, pl.ds(c * LANES, LANES)]
                            lr = jnp.maximum(z, 0.2 * z)
                            t = att_r[c] * lr
                            accv = t if accv is None else accv + t
                        for sh in (1, 2, 4, 8):
                            accv = accv + jnp.take(accv, lanes ^ sh)
                        ex = jnp.exp(accv)
                        for c in range(nslice):
                            msgb[e, pl.ds(c * LANES, LANES)] = ex * a[c]
                        idxv = jnp.take(
                            dvv, jnp.full((LANES,), eo, jnp.int32))
                        dsc = dvv[eo]
                        drow = dsc >> 7
                        dcol = pl.multiple_of(
                            ((dsc >> 4) & 7) * LANES, LANES)
                        sel = jnp.where(
                            (idxv & (LANES - 1)) == lanes, ex, 0.0)
                        db = (den0, den1)[eo & 1]
                        db[drow, pl.ds(dcol, LANES)] = (
                            db[drow, pl.ds(dcol, LANES)] + sel)
                    return gcar
                lax.fori_loop(0, nsub, group, 0)

            bufs = ((xl0, xr0, svo0, dvo0, dvb0, semg0),
                    (xl1, xr1, svo1, dvo1, dvb1, semg1))
            msgs = ((msg0, dsc0, semsc0), (msg1, dsc1, semsc1))

            def step(it, car):
                for p in (0, 1):
                    ch = 2 * it + p
                    xlb, xrb, svob, dvob, dvbb, semg = bufs[p]
                    (oxlb, oxrb, osvob, odvob, odvbb, osemg) = bufs[1 - p]
                    m = 1 - p
                    msgb, dscb, semsc = msgs[m]

                    @pl.when((ch % SK == 0) & (ch < NCH))
                    def _():
                        sup = ch >> 3
                        pltpu.make_async_copy(
                            srcs_hbm.at[wid, 0], si.at[0], semi).wait()
                        pltpu.make_async_copy(
                            srcs_hbm.at[wid, 0], di.at[0], semi).wait()

                        @pl.when(sup + 1 < NSUP)
                        def _():
                            spn = (sup + 1) & 1
                            pltpu.async_copy(
                                srcs_hbm.at[wid, sup + 1], si.at[spn], semi)
                            pltpu.async_copy(
                                dsts_hbm.at[wid, sup + 1], di.at[spn], semi)

                    @pl.when(ch < NCH)
                    def _():
                        issue(ch, xlb, xrb, svob, dvob, dvbb, semg)

                    @pl.when(ch >= 3)
                    def _():
                        pltpu.make_async_copy(
                            msgb, acc.at[dscb], semsc).wait()

                    @pl.when((ch >= 1) & (ch <= NCH))
                    def _():
                        pltpu.make_async_copy(xl_hbm.at[osvob],
                                              oxlb, osemg).wait()
                        pltpu.make_async_copy(xr_hbm.at[odvob],
                                              oxrb, osemg).wait()
                        compute(oxlb, oxrb, odvbb, msgb, dscb)
                        pltpu.async_copy(msgb, acc.at[dscb], semsc,
                                         add=True)
                return car
            lax.fori_loop(0, (NCH + 2) // 2, step, 0)
            pltpu.make_async_copy(msg1, acc.at[dsc1], semsc1).wait()

            def dcomb(r, car):
                for c in range(nslice):
                    sl = pl.ds(c * LANES, LANES)
                    den0[r, sl] = den0[r, sl] + den1[r, sl]
                return car
            lax.fori_loop(0, DR, dcomb, 0)
            pltpu.sync_copy(den0, dacc.at[riota], add=True)
            plsc.subcore_barrier()
            for k in range(RPT // EK):
                r0 = pl.multiple_of(sid * RPT, 64) + k * EK
                pltpu.sync_copy(acc.at[pl.ds(r0, EK)],
                                msg_out.at[cid, h, pl.ds(r0, EK)])

            @pl.when(sid < DR // 8)
            def _():
                d0 = pl.multiple_of(sid * 8, 8)
                pltpu.sync_copy(dacc.at[pl.ds(d0, 8)],
                                den_out.at[cid, h, pl.ds(d0, 8)])
            plsc.subcore_barrier()

            return hcar
        lax.fori_loop(0, H, head_body, 0)

    return gat_pass


def _proj_pass(x_pad, Wl, Wr, H, d_in):
    def body(x_ref, wl_ref, wr_ref, ol_ref, or_ref):
        xv = x_ref[...]
        ol_ref[...] = jnp.dot(xv, wl_ref[...],
                              preferred_element_type=jnp.float32)
        or_ref[...] = jnp.dot(xv, wr_ref[...],
                              preferred_element_type=jnp.float32)
    return pl.pallas_call(
        body,
        grid=(H, NT),
        in_specs=[
            pl.BlockSpec((TN, d_in), lambda h, i: (i, 0)),
            pl.BlockSpec((d_in, HID), lambda h, i: (0, h)),
            pl.BlockSpec((d_in, HID), lambda h, i: (0, h)),
        ],
        out_specs=[
            pl.BlockSpec((TN, HID), lambda h, i: (h * NT + i, 0)),
            pl.BlockSpec((TN, HID), lambda h, i: (h * NT + i, 0)),
        ],
        out_shape=[jax.ShapeDtypeStruct((H * NPAD, HID), jnp.float32)] * 2,
    )(x_pad, Wl, Wr)


def _norm_head(m_blk, d_blk, brow):
    d = jnp.sum(d_blk[:, 0, 0], axis=0)
    m3 = m_blk.reshape(TN // HID, HID, HID)
    h3 = m3 / (d[:, :, None] + 1e-16) + brow[None, None, :]
    return jnp.maximum(h3, 0.0).reshape(TN, HID)


def _combine_project(out1, den1, b1_2d, Wl2, Wr2):
    def body(m0, m1, m2, m3, d0, d1, d2, d3, b1_ref, wl_ref, wr_ref,
             ol_ref, or_ref):
        parts = []
        for hh, (m, d) in enumerate(((m0, d0), (m1, d1), (m2, d2),
                                     (m3, d3))):
            brow = b1_ref[0, hh * HID:(hh + 1) * HID]
            parts.append(_norm_head(m[0, 0] + m[1, 0], d, brow))
        h1 = jnp.concatenate(parts, axis=1)
        ol_ref[...] = jnp.dot(h1, wl_ref[...],
                              preferred_element_type=jnp.float32)
        or_ref[...] = jnp.dot(h1, wr_ref[...],
                              preferred_element_type=jnp.float32)

    msg_specs = [
        pl.BlockSpec((NC, 1, TN, HID), lambda i, hh=hh: (0, hh, i, 0))
        for hh in range(H1)
    ]
    den_specs = [
        pl.BlockSpec((NC, 1, 1, TN // HID, HID),
                     lambda i, hh=hh: (0, hh, i, 0, 0))
        for hh in range(H1)
    ]
    return pl.pallas_call(
        body,
        grid=(NT,),
        in_specs=msg_specs + den_specs + [
            pl.BlockSpec((1, H1 * HID), lambda i: (0, 0)),
            pl.BlockSpec((H1 * HID, HID), lambda i: (0, 0)),
            pl.BlockSpec((H1 * HID, HID), lambda i: (0, 0)),
        ],
        out_specs=[
            pl.BlockSpec((TN, HID), lambda i: (i, 0)),
            pl.BlockSpec((TN, HID), lambda i: (i, 0)),
        ],
        out_shape=[jax.ShapeDtypeStruct((NPAD, HID), jnp.float32)] * 2,
    )(out1, out1, out1, out1, den1, den1, den1, den1, b1_2d, Wl2, Wr2)


def _pool_pass(out2, den2, b2_2d, batch3):
    def body(m_ref, d_ref, b2_ref, bt_ref, ps_ref, cnt_ref):
        i = pl.program_id(0)
        h2 = _norm_head(m_ref[0, 0] + m_ref[1, 0], d_ref, b2_ref[0])
        bt = bt_ref[0]
        oh = (lax.broadcasted_iota(jnp.int32, (B, TN), 0) == bt
              ).astype(jnp.float32)
        ps = jnp.dot(oh, h2, preferred_element_type=jnp.float32)
        cnt = jnp.dot(oh, jnp.ones((TN, HID), jnp.float32),
                      preferred_element_type=jnp.float32)

        @pl.when(i == 0)
        def _():
            ps_ref[...] = jnp.zeros_like(ps_ref)
            cnt_ref[...] = jnp.zeros_like(cnt_ref)
        ps_ref[...] += ps
        cnt_ref[...] += cnt

    return pl.pallas_call(
        body,
        grid=(NT,),
        in_specs=[
            pl.BlockSpec((NC, 1, TN, HID), lambda i: (0, 0, i, 0)),
            pl.BlockSpec((NC, 1, 1, TN // HID, HID),
                         lambda i: (0, 0, i, 0, 0)),
            pl.BlockSpec((1, HID), lambda i: (0, 0)),
            pl.BlockSpec((1, 1, TN), lambda i: (i, 0, 0)),
        ],
        out_specs=[
            pl.BlockSpec((B, HID), lambda i: (0, 0)),
            pl.BlockSpec((B, HID), lambda i: (0, 0)),
        ],
        out_shape=[jax.ShapeDtypeStruct((B, HID), jnp.float32)] * 2,
    )(out2, den2, b2_2d, batch3)


def _head_pass(ps, cnt, doc_features, Wdoc, bdoc2, gamma2, beta2,
               Wfus, bfus2, Wtask, btask2, Wtime, btime2):
    def body(ps_ref, cnt_ref, docf_ref, wdoc_ref, bdoc_ref, g_ref, be_ref,
             wfus_ref, bfus_ref, wtask_ref, btask_ref, wtime_ref, btime_ref,
             task_ref, time_ref):
        pooled = ps_ref[...] / jnp.maximum(cnt_ref[...], 1.0)
        doc = jnp.maximum(
            jnp.dot(docf_ref[...], wdoc_ref[...],
                    preferred_element_type=jnp.float32) + bdoc_ref[...], 0.0)
        fusion = jnp.concatenate([pooled, doc], axis=1)
        mu = jnp.mean(fusion, axis=0, keepdims=True)
        var = jnp.mean((fusion - mu) ** 2, axis=0, keepdims=True)
        fusion = (fusion - mu) / jnp.sqrt(var + 1e-5) * g_ref[...] + be_ref[...]
        fusion = jnp.maximum(
            jnp.dot(fusion, wfus_ref[...],
                    preferred_element_type=jnp.float32) + bfus_ref[...], 0.0)
        task_ref[...] = jnp.dot(fusion, wtask_ref[...],
                                preferred_element_type=jnp.float32
                                ) + btask_ref[...]
        time_ref[...] = jnp.dot(fusion, wtime_ref[...],
                                preferred_element_type=jnp.float32
                                ) + btime_ref[...]

    return pl.pallas_call(
        body,
        out_shape=[jax.ShapeDtypeStruct((B, OUT), jnp.float32),
                   jax.ShapeDtypeStruct((B, 1), jnp.float32)],
    )(ps, cnt, doc_features, Wdoc, bdoc2, gamma2, beta2,
      Wfus, bfus2, Wtask, btask2, Wtime, btime2)


def kernel(x, edge_index, batch, doc_features, Wl1, Wr1, att1, b1,
           Wl2, Wr2, att2, b2, Wdoc, bdoc, gamma, beta, Wfus, bfus,
           Wtask, btask, Wtime, btime):
    x_pad = jnp.pad(x, ((0, NPAD - N), (0, 0)))
    loop = jnp.arange(N, dtype=jnp.int32)
    padi = jnp.full((EPAD - ETOT,), N, jnp.int32)
    srcs = jnp.concatenate([edge_index[0], loop, padi]).reshape(NW, NSUP, SKE)
    dsts = jnp.concatenate([edge_index[1], loop, padi]).reshape(NW, NSUP, SKE)
    batch3 = jnp.concatenate(
        [batch, jnp.full((NPAD - N,), B, jnp.int32)]).reshape(NT, 1, TN)

    xl1, xr1 = _proj_pass(x_pad, Wl1, Wr1, H1, D)
    out1, den1 = _gat_sc_pass(H1)(xl1, xr1, att1, srcs, dsts)
    den1 = den1.reshape(NC, H1, NT, TN // HID, HID)

    xl2, xr2 = _combine_project(out1, den1, b1.reshape(1, H1 * HID),
                                Wl2, Wr2)
    out2, den2 = _gat_sc_pass(1)(xl2, xr2, att2, srcs, dsts)
    den2 = den2.reshape(NC, 1, NT, TN // HID, HID)

    ps, cnt = _pool_pass(out2, den2, b2.reshape(1, HID), batch3)
    task, time = _head_pass(
        ps, cnt, doc_features, Wdoc, bdoc.reshape(1, HID),
        gamma.reshape(1, 2 * HID), beta.reshape(1, 2 * HID),
        Wfus, bfus.reshape(1, HID), Wtask, btask.reshape(1, OUT),
        Wtime, btime.reshape(1, 1))
    return (task, time)


# speedup vs baseline: 1.4382x; 1.4382x over previous
import functools

import jax
import jax.numpy as jnp
from jax import lax
from jax.experimental import pallas as pl
from jax.experimental.pallas import tpu as pltpu
from jax.experimental.pallas import tpu_sc as plsc

N = 10000
E = 320000
D = 128
H1 = 4
HID = 128
DOC = 256
B = 16
OUT = 10

NC, NS, LANES = 2, 16, 16
NW = NC * NS
NPAD = 10240
EK = 32
SK = 8
SKE = SK * EK
ETOT = E + N
NCH = 328
NSUP = NCH // SK
EPAD = NW * NCH * EK
RPT = NPAD // NS
DR = NPAD // HID
TN = 256
NT = NPAD // TN


def _gat_sc_pass(H):
    mesh = plsc.VectorSubcoreMesh(
        core_axis_name="c", subcore_axis_name="s",
        num_cores=NC, num_subcores=NS)

    @functools.partial(
        pl.kernel,
        out_type=[jax.ShapeDtypeStruct((NC, H, NPAD, HID), jnp.float32),
                  jax.ShapeDtypeStruct((NC, H, DR, HID), jnp.float32)],
        mesh=mesh,
        scratch_types=[
            pltpu.VMEM((EK, HID), jnp.float32),
            pltpu.VMEM((EK, HID), jnp.float32),
            pltpu.VMEM((EK, HID), jnp.float32),
            pltpu.VMEM((EK, HID), jnp.float32),
            pltpu.VMEM((EK, HID), jnp.float32),
            pltpu.VMEM((EK, HID), jnp.float32),
            pltpu.VMEM((2, SKE), jnp.int32),
            pltpu.VMEM((2, SKE), jnp.int32),
            pltpu.VMEM((EK,), jnp.int32),
            pltpu.VMEM((EK,), jnp.int32),
            pltpu.VMEM((EK,), jnp.int32),
            pltpu.VMEM((EK,), jnp.int32),
            pltpu.VMEM((EK,), jnp.int32),
            pltpu.VMEM((EK,), jnp.int32),
            pltpu.VMEM((EK,), jnp.int32),
            pltpu.VMEM((EK,), jnp.int32),
            pltpu.VMEM((HID,), jnp.float32),
            pltpu.VMEM((EK, HID), jnp.float32),
            pltpu.VMEM((EK, HID), jnp.float32),
            pltpu.VMEM((EK,), jnp.int32),
            pltpu.VMEM((EK,), jnp.int32),
            pltpu.VMEM_SHARED((NPAD, HID), jnp.float32),
            pltpu.VMEM_SHARED((DR, HID), jnp.float32),
            pltpu.SemaphoreType.DMA,
            pltpu.SemaphoreType.DMA,
            pltpu.SemaphoreType.DMA,
            pltpu.SemaphoreType.DMA,
            pltpu.SemaphoreType.DMA,
            pltpu.SemaphoreType.DMA,
            pltpu.SemaphoreType.DMA,
        ],
    )
    def gat_pass(xl_hbm, xr_hbm, att_hbm, srcs_hbm, dsts_hbm,
                 msg_out, den_out,
                 xl0, xl1, xr0, xr1, msg0, msg1, si, di,
                 svo0, svo1, dvo0, dvo1, dvb0, dvb1, dsc0, dsc1,
                 attv, exr0, exr1, dro0, dro1, acc, dacc,
                 semg0, semg1, semsc0, semsc1, semd0, semd1, semi):
        cid = lax.axis_index("c")
        sid = lax.axis_index("s")
        wid = cid * NS + sid
        nslice = HID // LANES
        nsub = EK // LANES

        def head_body(h, hcar):
            def zrow(r, car):
                for c in range(nslice):
                    msg0[r, pl.ds(c * LANES, LANES)] = jnp.zeros(
                        (LANES,), jnp.float32)
                return car
            lax.fori_loop(0, EK, zrow, 0)

            base = pl.multiple_of(sid * RPT, 64)
            for k in range(RPT // EK):
                pltpu.sync_copy(msg0, acc.at[pl.ds(base + k * EK, EK)])

            @pl.when(sid < DR // 8)
            def _():
                d0 = pl.multiple_of(sid * 8, 8)
                pltpu.sync_copy(msg0.at[pl.ds(0, 8)], dacc.at[pl.ds(d0, 8)])
            plsc.subcore_barrier()

            pltpu.sync_copy(att_hbm.at[h], attv)
            att_r = [attv[pl.ds(c * LANES, LANES)] for c in range(nslice)]

            pltpu.async_copy(srcs_hbm.at[wid, 0], si.at[0], semi)
            pltpu.async_copy(dsts_hbm.at[wid, 0], di.at[0], semi)

            def issue(ch, xlb, xrb, svob, dvob, dvbb, semg):
                sp = (ch >> 3) & 1
                cw = ch & (SK - 1)
                off = jnp.full((LANES,), h * NPAD, jnp.int32)
                for c in range(nsub):
                    sl = pl.ds(c * LANES, LANES)
                    s0 = pl.multiple_of(cw * EK + c * LANES, LANES)
                    sv = si[sp, pl.ds(s0, LANES)]
                    dv = di[sp, pl.ds(s0, LANES)]
                    svob[sl] = sv + off
                    dvob[sl] = dv + off
                    dvbb[sl] = dv
                c1 = pltpu.async_copy(xl_hbm.at[svob], xlb, semg)
                c2 = pltpu.async_copy(xr_hbm.at[dvob], xrb, semg)

            def compute(xlb, xrb, dvbb, msgb, dscb, exrb, drob):
                def group(g, gcar):
                    dvv = dvbb[pl.ds(g * LANES, LANES)]
                    lanes = lax.iota(jnp.int32, LANES)
                    dscb[pl.ds(g * LANES, LANES)] = dvv
                    drob[pl.ds(g * LANES, LANES)] = dvv >> 7
                    for eo in range(LANES):
                        e = g * LANES + eo
                        a = [xlb[e, pl.ds(c * LANES, LANES)]
                             for c in range(nslice)]
                        accv = None
                        for c in range(nslice):
                            z = a[c] + xrb[e, pl.ds(c * LANES, LANES)]
                            lr = jnp.maximum(z, 0.2 * z)
                            t = att_r[c] * lr
                            accv = t if accv is None else accv + t
                        for sh in (1, 2, 4, 8):
                            accv = accv + jnp.take(accv, lanes ^ sh)
                        ex = jnp.exp(accv)
                        for c in range(nslice):
                            msgb[e, pl.ds(c * LANES, LANES)] = ex * a[c]
                        idxv = jnp.take(
                            dvv, jnp.full((LANES,), eo, jnp.int32))
                        colv = idxv & (HID - 1)
                        for j in range(nslice):
                            v = jnp.where(colv == lanes + j * LANES,
                                          ex, 0.0)
                            exrb[e, pl.ds(j * LANES, LANES)] = v
                    return gcar
                lax.fori_loop(0, nsub, group, 0)

            bufs = ((xl0, xr0, svo0, dvo0, dvb0, semg0),
                    (xl1, xr1, svo1, dvo1, dvb1, semg1))
            msgs = ((msg0, dsc0, semsc0, exr0, dro0, semd0),
                    (msg1, dsc1, semsc1, exr1, dro1, semd1))

            def step(it, car):
                for p in (0, 1):
                    ch = 2 * it + p
                    xlb, xrb, svob, dvob, dvbb, semg = bufs[p]
                    (oxlb, oxrb, osvob, odvob, odvbb, osemg) = bufs[1 - p]
                    m = 1 - p
                    msgb, dscb, semsc, exrb, drob, semd = msgs[m]

                    @pl.when((ch % SK == 0) & (ch < NCH))
                    def _():
                        sup = ch >> 3
                        pltpu.make_async_copy(
                            srcs_hbm.at[wid, 0], si.at[0], semi).wait()
                        pltpu.make_async_copy(
                            srcs_hbm.at[wid, 0], di.at[0], semi).wait()

                        @pl.when(sup + 1 < NSUP)
                        def _():
                            spn = (sup + 1) & 1
                            pltpu.async_copy(
                                srcs_hbm.at[wid, sup + 1], si.at[spn], semi)
                            pltpu.async_copy(
                                dsts_hbm.at[wid, sup + 1], di.at[spn], semi)

                    @pl.when(ch < NCH)
                    def _():
                        issue(ch, xlb, xrb, svob, dvob, dvbb, semg)

                    @pl.when(ch >= 3)
                    def _():
                        pltpu.make_async_copy(
                            msgb, acc.at[dscb], semsc).wait()
                        pltpu.make_async_copy(
                            exrb, dacc.at[drob], semd).wait()

                    @pl.when((ch >= 1) & (ch <= NCH))
                    def _():
                        pltpu.make_async_copy(xl_hbm.at[osvob],
                                              oxlb, osemg).wait()
                        pltpu.make_async_copy(xr_hbm.at[odvob],
                                              oxrb, osemg).wait()
                        compute(oxlb, oxrb, odvbb, msgb, dscb, exrb, drob)
                        pltpu.async_copy(msgb, acc.at[dscb], semsc,
                                         add=True)
                        pltpu.async_copy(exrb, dacc.at[drob], semd,
                                         add=True)
                return car
            lax.fori_loop(0, (NCH + 2) // 2, step, 0)
            pltpu.make_async_copy(msg1, acc.at[dsc1], semsc1).wait()

            pltpu.make_async_copy(exr1, dacc.at[dro1], semd1).wait()
            plsc.subcore_barrier()
            for k in range(RPT // EK):
                r0 = pl.multiple_of(sid * RPT, 64) + k * EK
                pltpu.sync_copy(acc.at[pl.ds(r0, EK)],
                                msg_out.at[cid, h, pl.ds(r0, EK)])

            @pl.when(sid < DR // 8)
            def _():
                d0 = pl.multiple_of(sid * 8, 8)
                pltpu.sync_copy(dacc.at[pl.ds(d0, 8)],
                                den_out.at[cid, h, pl.ds(d0, 8)])
            plsc.subcore_barrier()

            return hcar
        lax.fori_loop(0, H, head_body, 0)

    return gat_pass


def _proj_pass(x_pad, Wl, Wr, H, d_in):
    def body(x_ref, wl_ref, wr_ref, ol_ref, or_ref):
        xv = x_ref[...]
        ol_ref[...] = jnp.dot(xv, wl_ref[...],
                              preferred_element_type=jnp.float32)
        or_ref[...] = jnp.dot(xv, wr_ref[...],
                              preferred_element_type=jnp.float32)
    return pl.pallas_call(
        body,
        grid=(H, NT),
        in_specs=[
            pl.BlockSpec((TN, d_in), lambda h, i: (i, 0)),
            pl.BlockSpec((d_in, HID), lambda h, i: (0, h)),
            pl.BlockSpec((d_in, HID), lambda h, i: (0, h)),
        ],
        out_specs=[
            pl.BlockSpec((TN, HID), lambda h, i: (h * NT + i, 0)),
            pl.BlockSpec((TN, HID), lambda h, i: (h * NT + i, 0)),
        ],
        out_shape=[jax.ShapeDtypeStruct((H * NPAD, HID), jnp.float32)] * 2,
    )(x_pad, Wl, Wr)


def _norm_head(m_blk, d_blk, brow):
    d = jnp.sum(d_blk[:, 0, 0], axis=0)
    m3 = m_blk.reshape(TN // HID, HID, HID)
    h3 = m3 / (d[:, :, None] + 1e-16) + brow[None, None, :]
    return jnp.maximum(h3, 0.0).reshape(TN, HID)


def _combine_project(out1, den1, b1_2d, Wl2, Wr2):
    def body(m0, m1, m2, m3, d0, d1, d2, d3, b1_ref, wl_ref, wr_ref,
             ol_ref, or_ref):
        parts = []
        for hh, (m, d) in enumerate(((m0, d0), (m1, d1), (m2, d2),
                                     (m3, d3))):
            brow = b1_ref[0, hh * HID:(hh + 1) * HID]
            parts.append(_norm_head(m[0, 0] + m[1, 0], d, brow))
        h1 = jnp.concatenate(parts, axis=1)
        ol_ref[...] = jnp.dot(h1, wl_ref[...],
                              preferred_element_type=jnp.float32)
        or_ref[...] = jnp.dot(h1, wr_ref[...],
                              preferred_element_type=jnp.float32)

    msg_specs = [
        pl.BlockSpec((NC, 1, TN, HID), lambda i, hh=hh: (0, hh, i, 0))
        for hh in range(H1)
    ]
    den_specs = [
        pl.BlockSpec((NC, 1, 1, TN // HID, HID),
                     lambda i, hh=hh: (0, hh, i, 0, 0))
        for hh in range(H1)
    ]
    return pl.pallas_call(
        body,
        grid=(NT,),
        in_specs=msg_specs + den_specs + [
            pl.BlockSpec((1, H1 * HID), lambda i: (0, 0)),
            pl.BlockSpec((H1 * HID, HID), lambda i: (0, 0)),
            pl.BlockSpec((H1 * HID, HID), lambda i: (0, 0)),
        ],
        out_specs=[
            pl.BlockSpec((TN, HID), lambda i: (i, 0)),
            pl.BlockSpec((TN, HID), lambda i: (i, 0)),
        ],
        out_shape=[jax.ShapeDtypeStruct((NPAD, HID), jnp.float32)] * 2,
    )(out1, out1, out1, out1, den1, den1, den1, den1, b1_2d, Wl2, Wr2)


def _pool_pass(out2, den2, b2_2d, batch3):
    def body(m_ref, d_ref, b2_ref, bt_ref, ps_ref, cnt_ref):
        i = pl.program_id(0)
        h2 = _norm_head(m_ref[0, 0] + m_ref[1, 0], d_ref, b2_ref[0])
        bt = bt_ref[0]
        oh = (lax.broadcasted_iota(jnp.int32, (B, TN), 0) == bt
              ).astype(jnp.float32)
        ps = jnp.dot(oh, h2, preferred_element_type=jnp.float32)
        cnt = jnp.dot(oh, jnp.ones((TN, HID), jnp.float32),
                      preferred_element_type=jnp.float32)

        @pl.when(i == 0)
        def _():
            ps_ref[...] = jnp.zeros_like(ps_ref)
            cnt_ref[...] = jnp.zeros_like(cnt_ref)
        ps_ref[...] += ps
        cnt_ref[...] += cnt

    return pl.pallas_call(
        body,
        grid=(NT,),
        in_specs=[
            pl.BlockSpec((NC, 1, TN, HID), lambda i: (0, 0, i, 0)),
            pl.BlockSpec((NC, 1, 1, TN // HID, HID),
                         lambda i: (0, 0, i, 0, 0)),
            pl.BlockSpec((1, HID), lambda i: (0, 0)),
            pl.BlockSpec((1, 1, TN), lambda i: (i, 0, 0)),
        ],
        out_specs=[
            pl.BlockSpec((B, HID), lambda i: (0, 0)),
            pl.BlockSpec((B, HID), lambda i: (0, 0)),
        ],
        out_shape=[jax.ShapeDtypeStruct((B, HID), jnp.float32)] * 2,
    )(out2, den2, b2_2d, batch3)


def _head_pass(ps, cnt, doc_features, Wdoc, bdoc2, gamma2, beta2,
               Wfus, bfus2, Wtask, btask2, Wtime, btime2):
    def body(ps_ref, cnt_ref, docf_ref, wdoc_ref, bdoc_ref, g_ref, be_ref,
             wfus_ref, bfus_ref, wtask_ref, btask_ref, wtime_ref, btime_ref,
             task_ref, time_ref):
        pooled = ps_ref[...] / jnp.maximum(cnt_ref[...], 1.0)
        doc = jnp.maximum(
            jnp.dot(docf_ref[...], wdoc_ref[...],
                    preferred_element_type=jnp.float32) + bdoc_ref[...], 0.0)
        fusion = jnp.concatenate([pooled, doc], axis=1)
        mu = jnp.mean(fusion, axis=0, keepdims=True)
        var = jnp.mean((fusion - mu) ** 2, axis=0, keepdims=True)
        fusion = (fusion - mu) / jnp.sqrt(var + 1e-5) * g_ref[...] + be_ref[...]
        fusion = jnp.maximum(
            jnp.dot(fusion, wfus_ref[...],
                    preferred_element_type=jnp.float32) + bfus_ref[...], 0.0)
        task_ref[...] = jnp.dot(fusion, wtask_ref[...],
                                preferred_element_type=jnp.float32
                                ) + btask_ref[...]
        time_ref[...] = jnp.dot(fusion, wtime_ref[...],
                                preferred_element_type=jnp.float32
                                ) + btime_ref[...]

    return pl.pallas_call(
        body,
        out_shape=[jax.ShapeDtypeStruct((B, OUT), jnp.float32),
                   jax.ShapeDtypeStruct((B, 1), jnp.float32)],
    )(ps, cnt, doc_features, Wdoc, bdoc2, gamma2, beta2,
      Wfus, bfus2, Wtask, btask2, Wtime, btime2)


def kernel(x, edge_index, batch, doc_features, Wl1, Wr1, att1, b1,
           Wl2, Wr2, att2, b2, Wdoc, bdoc, gamma, beta, Wfus, bfus,
           Wtask, btask, Wtime, btime):
    x_pad = jnp.pad(x, ((0, NPAD - N), (0, 0)))
    loop = jnp.arange(N, dtype=jnp.int32)
    padi = jnp.full((EPAD - ETOT,), N, jnp.int32)
    srcs = jnp.concatenate([edge_index[0], loop, padi]).reshape(NW, NSUP, SKE)
    dsts = jnp.concatenate([edge_index[1], loop, padi]).reshape(NW, NSUP, SKE)
    batch3 = jnp.concatenate(
        [batch, jnp.full((NPAD - N,), B, jnp.int32)]).reshape(NT, 1, TN)

    xl1, xr1 = _proj_pass(x_pad, Wl1, Wr1, H1, D)
    out1, den1 = _gat_sc_pass(H1)(xl1, xr1, att1, srcs, dsts)
    den1 = den1.reshape(NC, H1, NT, TN // HID, HID)

    xl2, xr2 = _combine_project(out1, den1, b1.reshape(1, H1 * HID),
                                Wl2, Wr2)
    out2, den2 = _gat_sc_pass(1)(xl2, xr2, att2, srcs, dsts)
    den2 = den2.reshape(NC, 1, NT, TN // HID, HID)

    ps, cnt = _pool_pass(out2, den2, b2.reshape(1, HID), batch3)
    task, time = _head_pass(
        ps, cnt, doc_features, Wdoc, bdoc.reshape(1, HID),
        gamma.reshape(1, 2 * HID), beta.reshape(1, 2 * HID),
        Wfus, bfus.reshape(1, HID), Wtask, btask.reshape(1, OUT),
        Wtime, btime.reshape(1, 1))
    return (task, time)
